# Initial kernel scaffold; baseline (speedup 1.0000x reference)
import jax
import jax.numpy as jnp
from jax.experimental import pallas as pl


def kernel(x, Wi, bi, Wo, bo, jump_destinations, Wg, bg, context_strength, pointer_init):
    raise NotImplementedError("write your pallas kernel here")


# speedup vs baseline: 1.7344x; 1.7344x over previous
import jax
import jax.numpy as jnp
from jax import lax
from jax.experimental import pallas as pl
from jax.experimental.pallas import tpu as pltpu

B, T, P, D, K = 256, 32, 512, 32, 2
TEMP = 8.0
NW = 2 * K + 1
BB = 64
G = B // BB
PC = 8
CP = P // PC


def _body(xT_ref, Wi_ref, bi_ref, Wo_ref, bo_ref, jd_ref, Wg_ref, bg_ref,
          cs_ref, p0_ref, out_ref, ring_ref):
    ring_ref[...] = jnp.zeros((BB, P, D), jnp.float32)
    WiT = Wi_ref[...].T
    WoT = Wo_ref[...].T
    bi = bi_ref[...]
    bo = bo_ref[...]
    bg = bg_ref[0, 0]
    jd = jd_ref[...]
    cscale = jax.nn.sigmoid(cs_ref[0, 0])
    p_iota_i = lax.broadcasted_iota(jnp.int32, (BB, P), 1)
    p_iota_f = p_iota_i.astype(jnp.float32)

    def step(t, carry):
        ptr, hidden = carry
        xt = xT_ref[t]
        iv = jnp.dot(xt, WiT, preferred_element_type=jnp.float32) + bi
        base = jnp.clip(jnp.floor(ptr), 0.0, P - 1.0).astype(jnp.int32)
        offm = jnp.remainder(p_iota_i - base + K, P)
        mask = (offm < NW).astype(jnp.float32)
        delta = jnp.remainder(p_iota_f - ptr + P / 2.0, float(P)) - P / 2.0
        e = jnp.exp(-(delta * delta) * (1.0 / TEMP)) * mask
        w = e / jnp.sum(e, axis=1, keepdims=True)

        ctx = jnp.zeros((BB, D), jnp.float32)
        for c in range(PC):
            rc = ring_ref[:, c * CP:(c + 1) * CP, :]
            wc = w[:, c * CP:(c + 1) * CP]
            ctx = ctx + jnp.sum(wc[:, :, None] * rc, axis=1)

        state = jnp.tanh(iv + cscale * ctx + hidden)

        for c in range(PC):
            wc = w[:, c * CP:(c + 1) * CP]
            ring_ref[:, c * CP:(c + 1) * CP, :] += wc[:, :, None] * state[:, None, :]

        cur = jnp.clip(ptr.astype(jnp.int32), 0, P - 1)
        jt = jnp.sum(jnp.where(p_iota_i == cur, jd, 0.0), axis=1, keepdims=True)
        zl = jnp.sum(state * Wg_ref[...], axis=1, keepdims=True) + bg
        walk = jnp.remainder(ptr + 1.0, float(P))
        ptr_new = jnp.where(zl > 0.0, jt, walk)
        out_ref[t] = jnp.dot(state, WoT, preferred_element_type=jnp.float32) + bo
        return ptr_new, state

    lax.fori_loop(0, T, step,
                  (p0_ref[...], jnp.zeros((BB, D), jnp.float32)))


@jax.jit
def kernel(x, Wi, bi, Wo, bo, jump_destinations, Wg, bg, context_strength,
           pointer_init):
    xT = jnp.swapaxes(x, 0, 1)
    outT = pl.pallas_call(
        _body,
        grid=(G,),
        in_specs=[
            pl.BlockSpec((T, BB, 8), lambda g: (0, g, 0)),
            pl.BlockSpec((D, 8), lambda g: (0, 0)),
            pl.BlockSpec((1, D), lambda g: (0, 0)),
            pl.BlockSpec((8, D), lambda g: (0, 0)),
            pl.BlockSpec((1, 8), lambda g: (0, 0)),
            pl.BlockSpec((1, P), lambda g: (0, 0)),
            pl.BlockSpec((1, D), lambda g: (0, 0)),
            pl.BlockSpec((1, 1), lambda g: (0, 0)),
            pl.BlockSpec((1, 1), lambda g: (0, 0)),
            pl.BlockSpec((BB, 1), lambda g: (g, 0)),
        ],
        out_specs=pl.BlockSpec((T, BB, 8), lambda g: (0, g, 0)),
        out_shape=jax.ShapeDtypeStruct((T, B, 8), jnp.float32),
        scratch_shapes=[pltpu.VMEM((BB, P, D), jnp.float32)],
    )(xT, Wi, bi.reshape(1, D), Wo, bo.reshape(1, 8),
      jump_destinations.reshape(1, P), Wg, bg.reshape(1, 1),
      context_strength.reshape(1, 1), pointer_init.reshape(B, 1))
    return jnp.swapaxes(outT, 0, 1)
